# baseline (speedup 1.0000x reference)
import functools

import jax
import jax.numpy as jnp
from jax import lax
from jax.experimental import pallas as pl
from jax.experimental.pallas import tpu as pltpu
from jax.experimental.pallas import tpu_sc as plsc

_NC = 26
_UPC = 126
_NU = _NC * _UPC
_CNT = 105
_DCAP = 2080


def _iota16():
    return lax.broadcasted_iota(jnp.int32, (16,), 0)


def _scan_set(buf, vals, dirty, g, ci, half, lo):
    ones = jnp.full((16,), 1.0, jnp.float32)

    def _it(i, cnt):
        for k in range(4):
            gi = half * 128 + i * 4 + k
            v = vals[ci, pl.ds(gi * 16, 16)]
            n16 = gi * 16 + _iota16()
            u = (v - lo).astype(jnp.uint32)
            hit = u < 16
            slot = (v >> 3) & 1
            m16 = (n16 >> 7) & 15
            s16 = v & 7
            l16 = n16 & 127
            plsc.store_scatter(buf, [2 * g + slot, m16, s16, l16], ones,
                               mask=hit)
            f16 = (slot << 14) | (m16 << 10) | (s16 << 7) | l16
            plsc.store_compressed(dirty.at[g, pl.ds(cnt, 16)], f16, mask=hit)
            cnt = cnt + plsc.all_reduce_population_count(hit)[0]
        return cnt

    return lax.fori_loop(0, 32, _it, jnp.int32(0))


def _clear(buf, dirty, g, cntg):
    zf = jnp.zeros((16,), jnp.float32)

    def _it(k, _):
        fv = dirty[g, pl.ds(k * 16, 16)]
        live = (k * 16 + _iota16()) < cntg
        plsc.store_scatter(
            buf,
            [2 * g + ((fv >> 14) & 1), (fv >> 10) & 15, (fv >> 7) & 7,
             fv & 127],
            zf, mask=live)
        return 0

    lax.fori_loop(0, (cntg + 15) >> 4, _it, 0)


def _sc_body(xt_hbm, out_hbm, vals, buf, dirty,
             sem0, sem1, sem2, sem3, sem4, sem5):
    wid = lax.axis_index("s") * 2 + lax.axis_index("c")
    base = (wid * (_NU - _CNT)) // 31
    c0 = base // _UPC
    c1 = jnp.minimum(c0 + 1, _NC - 1)

    pltpu.sync_copy(xt_hbm.at[c0], vals.at[0])
    pltpu.sync_copy(xt_hbm.at[c1], vals.at[1])

    def _zb(u, _):
        sl, m = u >> 4, u & 15
        for s in range(8):
            for l0 in range(0, 128, 16):
                buf[sl, m, s, pl.ds(l0, 16)] = jnp.zeros((16,), jnp.float32)
        return 0

    lax.fori_loop(0, 96, _zb, 0)

    sems = ((sem0, sem1), (sem2, sem3), (sem4, sem5))

    def _locate(u):
        q = base + u
        c = q // _UPC
        r = q - c * _UPC
        tp = r >> 1
        return c, tp, r & 1

    def _fire(g, u):
        c, tp, half = _locate(u)
        cnt = _scan_set(buf, vals, dirty, g, c - c0, half, tp * 16)
        t0 = 2 * tp
        nt = 16 * half
        pltpu.async_copy(buf.at[2 * g], out_hbm.at[c, t0, pl.ds(nt, 16)],
                         sems[g][0])
        dummy = tp == 62
        @pl.when(jnp.logical_not(dummy))
        def _():
            pltpu.async_copy(buf.at[2 * g + 1],
                             out_hbm.at[c, t0 + 1, pl.ds(nt, 16)], sems[g][1])
        @pl.when(dummy)
        def _():
            pltpu.async_copy(buf.at[2 * g],
                             out_hbm.at[c, t0, pl.ds(nt, 16)], sems[g][1])
        return cnt

    cnt0 = _fire(0, 0)
    cnt1 = _fire(1, 1)
    cnt2 = _fire(2, 2)

    def _dummy_win():
        return out_hbm.at[0, 0, pl.ds(0, 16)]

    def _pair(p, carry):
        new = list(carry)
        for g in range(3):
            u = 3 * p + g
            for s in range(2):
                pltpu.make_async_copy(
                    buf.at[2 * g + s], _dummy_win(), sems[g][s]).wait()
            _clear(buf, dirty, g, new[g])
            new[g] = _fire(g, u)
        return tuple(new)

    lax.fori_loop(1, _CNT // 3, _pair, (cnt0, cnt1, cnt2))

    for g in range(3):
        for s in range(2):
            pltpu.make_async_copy(
                buf.at[2 * g + s], _dummy_win(), sems[g][s]).wait()


_sc_one_hot = functools.partial(
    pl.kernel,
    out_type=jax.ShapeDtypeStruct((_NC, 125, 32, 8, 128), jnp.float32),
    mesh=plsc.VectorSubcoreMesh(core_axis_name="c", subcore_axis_name="s"),
    scratch_types=[
        pltpu.VMEM((2, 4096), jnp.int32),
        pltpu.VMEM((6, 16, 8, 128), jnp.float32),
        pltpu.VMEM((3, _DCAP), jnp.int32),
        pltpu.SemaphoreType.DMA,
        pltpu.SemaphoreType.DMA,
        pltpu.SemaphoreType.DMA,
        pltpu.SemaphoreType.DMA,
        pltpu.SemaphoreType.DMA,
        pltpu.SemaphoreType.DMA,
    ],
    compiler_params=pltpu.CompilerParams(needs_layout_passes=False),
)(_sc_body)


def kernel(inputs):
    xt = inputs.astype(jnp.int32).T
    out5 = _sc_one_hot(xt)
    return out5.transpose(2, 4, 0, 1, 3).reshape(4096, 26, 1000)


# speedup vs baseline: 1.0147x; 1.0147x over previous
import functools

import jax
import jax.numpy as jnp
from jax import lax
from jax.experimental import pallas as pl
from jax.experimental.pallas import tpu as pltpu
from jax.experimental.pallas import tpu_sc as plsc

_NC = 26
_UPC = 126
_NU = _NC * _UPC
_CNT = 104
_DCAP = 2080


def _iota16():
    return lax.broadcasted_iota(jnp.int32, (16,), 0)


def _scan_set(buf, vals, dirty, g, ci, half, lo):
    ones = jnp.full((16,), 1.0, jnp.float32)

    def _it(i, cnt):
        for k in range(4):
            gi = half * 128 + i * 4 + k
            v = vals[ci, pl.ds(gi * 16, 16)]
            n16 = gi * 16 + _iota16()
            u = (v - lo).astype(jnp.uint32)
            hit = u < 16
            slot = (v >> 3) & 1
            m16 = (n16 >> 7) & 15
            s16 = v & 7
            l16 = n16 & 127
            plsc.store_scatter(buf, [2 * g + slot, m16, s16, l16], ones,
                               mask=hit)
            f16 = (slot << 14) | (m16 << 10) | (s16 << 7) | l16
            plsc.store_compressed(dirty.at[g, pl.ds(cnt, 16)], f16, mask=hit)
            cnt = cnt + plsc.all_reduce_population_count(hit)[0]
        return cnt

    return lax.fori_loop(0, 32, _it, jnp.int32(0))


def _clear(buf, dirty, g, cntg):
    zf = jnp.zeros((16,), jnp.float32)

    def _it(k, _):
        fv = dirty[g, pl.ds(k * 16, 16)]
        live = (k * 16 + _iota16()) < cntg
        plsc.store_scatter(
            buf,
            [2 * g + ((fv >> 14) & 1), (fv >> 10) & 15, (fv >> 7) & 7,
             fv & 127],
            zf, mask=live)
        return 0

    lax.fori_loop(0, (cntg + 15) >> 4, _it, 0)


def _sc_body(xt_hbm, out_hbm, vals, buf, dirty, sem0, sem1, sem2, sem3):
    wid = lax.axis_index("s") * 2 + lax.axis_index("c")
    base = (wid * (_NU - _CNT)) // 31
    c0 = base // _UPC
    c1 = jnp.minimum(c0 + 1, _NC - 1)

    pltpu.sync_copy(xt_hbm.at[c0], vals.at[0])
    pltpu.sync_copy(xt_hbm.at[c1], vals.at[1])

    def _zb(u, _):
        sl, m = u >> 4, u & 15
        for s in range(8):
            for l0 in range(0, 128, 16):
                buf[sl, m, s, pl.ds(l0, 16)] = jnp.zeros((16,), jnp.float32)
        return 0

    lax.fori_loop(0, 64, _zb, 0)

    def _zd(u, _):
        for g in range(2):
            dirty[g, pl.ds(u * 16, 16)] = jnp.zeros((16,), jnp.int32)
        return 0

    lax.fori_loop(0, _DCAP // 16, _zd, 0)

    sems = ((sem0, sem1), (sem2, sem3))

    def _locate(u):
        q = base + u
        c = q // _UPC
        r = q - c * _UPC
        tp = r >> 1
        return c, tp, r & 1

    def _fire(g, u):
        c, tp, half = _locate(u)
        cnt = _scan_set(buf, vals, dirty, g, c - c0, half, tp * 16)
        t0 = 2 * tp
        nt = 16 * half
        pltpu.async_copy(buf.at[2 * g], out_hbm.at[c, t0, pl.ds(nt, 16)],
                         sems[g][0])
        dummy = tp == 62
        @pl.when(jnp.logical_not(dummy))
        def _():
            pltpu.async_copy(buf.at[2 * g + 1],
                             out_hbm.at[c, t0 + 1, pl.ds(nt, 16)], sems[g][1])
        @pl.when(dummy)
        def _():
            pltpu.async_copy(buf.at[2 * g],
                             out_hbm.at[c, t0, pl.ds(nt, 16)], sems[g][1])
        return cnt

    cnt0 = _fire(0, 0)
    cnt1 = _fire(1, 1)

    def _dummy_win():
        return out_hbm.at[0, 0, pl.ds(0, 16)]

    def _pair(p, carry):
        new = list(carry)
        for g in range(2):
            u = 2 * p + g
            for s in range(2):
                pltpu.make_async_copy(
                    buf.at[2 * g + s], _dummy_win(), sems[g][s]).wait()
            _clear(buf, dirty, g, new[g])
            new[g] = _fire(g, u)
        return tuple(new)

    lax.fori_loop(1, _CNT // 2, _pair, (cnt0, cnt1))

    for g in range(2):
        for s in range(2):
            pltpu.make_async_copy(
                buf.at[2 * g + s], _dummy_win(), sems[g][s]).wait()


_sc_one_hot = functools.partial(
    pl.kernel,
    out_type=jax.ShapeDtypeStruct((_NC, 125, 32, 8, 128), jnp.float32),
    mesh=plsc.VectorSubcoreMesh(core_axis_name="c", subcore_axis_name="s"),
    scratch_types=[
        pltpu.VMEM((2, 4096), jnp.int32),
        pltpu.VMEM((4, 16, 8, 128), jnp.float32),
        pltpu.VMEM((2, _DCAP), jnp.int32),
        pltpu.SemaphoreType.DMA,
        pltpu.SemaphoreType.DMA,
        pltpu.SemaphoreType.DMA,
        pltpu.SemaphoreType.DMA,
    ],
    compiler_params=pltpu.CompilerParams(needs_layout_passes=False),
)(_sc_body)


def kernel(inputs):
    xt = inputs.astype(jnp.int32).T
    out5 = _sc_one_hot(xt)
    return out5.transpose(2, 4, 0, 1, 3).reshape(4096, 26, 1000)


# speedup vs baseline: 1.0229x; 1.0081x over previous
import functools

import jax
import jax.numpy as jnp
from jax import lax
from jax.experimental import pallas as pl
from jax.experimental.pallas import tpu as pltpu
from jax.experimental.pallas import tpu_sc as plsc

_NC = 26
_UPC = 126
_NU = _NC * _UPC
_CNT = 104
_DCAP = 2080


def _iota16():
    return lax.broadcasted_iota(jnp.int32, (16,), 0)


def _scan_set(buf, vals, dirty, g, ci, half, lo):
    ones = jnp.full((16,), 1.0, jnp.float32)

    def _it(i, cnt):
        for k in range(4):
            gi = half * 128 + i * 4 + k
            v = vals[ci, pl.ds(gi * 16, 16)]
            n16 = gi * 16 + _iota16()
            u = (v - lo).astype(jnp.uint32)
            hit = u < 16
            slot = (v >> 3) & 1
            m16 = (n16 >> 7) & 15
            s16 = v & 7
            l16 = n16 & 127
            plsc.store_scatter(buf, [2 * g + slot, m16, s16, l16], ones,
                               mask=hit)
            f16 = (slot << 14) | (m16 << 10) | (s16 << 7) | l16
            plsc.store_compressed(dirty.at[g, pl.ds(cnt, 16)], f16, mask=hit)
            cnt = cnt + plsc.all_reduce_population_count(hit)[0]
        return cnt

    return lax.fori_loop(0, 32, _it, jnp.int32(0))


def _clear(buf, dirty, g, cntg):
    zf = jnp.zeros((16,), jnp.float32)

    def _it(k, _):
        fv = dirty[g, pl.ds(k * 16, 16)]
        live = (k * 16 + _iota16()) < cntg
        plsc.store_scatter(
            buf,
            [2 * g + ((fv >> 14) & 1), (fv >> 10) & 15, (fv >> 7) & 7,
             fv & 127],
            zf, mask=live)
        return 0

    lax.fori_loop(0, (cntg + 15) >> 4, _it, 0)


def _sc_body(xt_hbm, out_hbm, vals, buf, dirty, sem0, sem1, sem2, sem3):
    wid = lax.axis_index("s") * 2 + lax.axis_index("c")
    base = (wid * (_NU - _CNT)) // 31
    c0 = base // _UPC
    c1 = jnp.minimum(c0 + 1, _NC - 1)

    stage0 = pltpu.async_copy(xt_hbm.at[c0], vals.at[0], sem0)
    stage1 = pltpu.async_copy(xt_hbm.at[c1], vals.at[1], sem1)

    def _zb(u, _):
        sl, m = u >> 4, u & 15
        for s in range(8):
            for l0 in range(0, 128, 16):
                buf[sl, m, s, pl.ds(l0, 16)] = jnp.zeros((16,), jnp.float32)
        return 0

    lax.fori_loop(0, 64, _zb, 0)
    stage0.wait()
    stage1.wait()

    def _zd(u, _):
        for g in range(2):
            dirty[g, pl.ds(u * 16, 16)] = jnp.zeros((16,), jnp.int32)
        return 0

    lax.fori_loop(0, _DCAP // 16, _zd, 0)

    sems = ((sem0, sem1), (sem2, sem3))

    def _locate(u):
        q = base + u
        c = q // _UPC
        r = q - c * _UPC
        tp = r >> 1
        return c, tp, r & 1

    def _fire(g, u):
        c, tp, half = _locate(u)
        cnt = _scan_set(buf, vals, dirty, g, c - c0, half, tp * 16)
        t0 = 2 * tp
        nt = 16 * half
        pltpu.async_copy(buf.at[2 * g], out_hbm.at[c, t0, pl.ds(nt, 16)],
                         sems[g][0])
        dummy = tp == 62
        @pl.when(jnp.logical_not(dummy))
        def _():
            pltpu.async_copy(buf.at[2 * g + 1],
                             out_hbm.at[c, t0 + 1, pl.ds(nt, 16)], sems[g][1])
        @pl.when(dummy)
        def _():
            pltpu.async_copy(buf.at[2 * g],
                             out_hbm.at[c, t0, pl.ds(nt, 16)], sems[g][1])
        return cnt

    cnt0 = _fire(0, 0)
    cnt1 = _fire(1, 1)

    def _dummy_win():
        return out_hbm.at[0, 0, pl.ds(0, 16)]

    def _pair(p, carry):
        new = list(carry)
        for g in range(2):
            u = 2 * p + g
            for s in range(2):
                pltpu.make_async_copy(
                    buf.at[2 * g + s], _dummy_win(), sems[g][s]).wait()
            _clear(buf, dirty, g, new[g])
            new[g] = _fire(g, u)
        return tuple(new)

    lax.fori_loop(1, _CNT // 2, _pair, (cnt0, cnt1))

    for g in range(2):
        for s in range(2):
            pltpu.make_async_copy(
                buf.at[2 * g + s], _dummy_win(), sems[g][s]).wait()


_sc_one_hot = functools.partial(
    pl.kernel,
    out_type=jax.ShapeDtypeStruct((_NC, 125, 32, 8, 128), jnp.float32),
    mesh=plsc.VectorSubcoreMesh(core_axis_name="c", subcore_axis_name="s"),
    scratch_types=[
        pltpu.VMEM((2, 4096), jnp.int32),
        pltpu.VMEM((4, 16, 8, 128), jnp.float32),
        pltpu.VMEM((2, _DCAP), jnp.int32),
        pltpu.SemaphoreType.DMA,
        pltpu.SemaphoreType.DMA,
        pltpu.SemaphoreType.DMA,
        pltpu.SemaphoreType.DMA,
    ],
    compiler_params=pltpu.CompilerParams(needs_layout_passes=False),
)(_sc_body)


def kernel(inputs):
    xt = inputs.astype(jnp.int32).T
    out5 = _sc_one_hot(xt)
    return out5.transpose(2, 4, 0, 1, 3).reshape(4096, 26, 1000)
